# baseline (speedup 1.0000x reference)
import functools

import jax
import jax.numpy as jnp
from jax.experimental import pallas as pl

BF = jnp.bfloat16


def _deg_cast_kernel(adj_ref, deg_ref, adjc_ref, *, n, s_blk, nk):
    k = pl.program_id(1)

    @pl.when(k == 0)
    def _init():
        deg_ref[...] = jnp.ones_like(deg_ref)

    @pl.when(k < nk - 1)
    def _full():
        blk = adj_ref[...]
        adjc_ref[...] = blk.astype(BF)
        deg_ref[...] += jnp.sum(blk, axis=0, keepdims=True)

    @pl.when(k == nk - 1)
    def _tail():
        blk = adj_ref[...]
        rows = jax.lax.broadcasted_iota(jnp.int32, blk.shape, 0)
        blk = jnp.where(rows < n - (nk - 1) * s_blk, blk, 0.0)
        adjc_ref[...] = blk.astype(BF)
        deg_ref[...] += jnp.sum(blk, axis=0, keepdims=True)


def _proj_kernel(ht_ref, w_ref, u_ref, pt_ref, hi_ref, *, n, s_blk):
    r = pl.program_id(0)
    pt = u_ref[...] * jax.lax.dot_general(
        w_ref[...],
        ht_ref[...],
        (((0,), (0,)), ((), ())),
        preferred_element_type=jnp.float32,
    )
    cols = jax.lax.broadcasted_iota(jnp.int32, pt.shape, 1)
    pt = jnp.where(cols < n - r * s_blk, pt, 0.0)
    pt_ref[...] = pt
    hi_ref[...] = pt.astype(BF)


def _conv_kernel(adjc_ref, hi_ref, pself_ref, u_ref, b_ref, o_ref, *, nk, act):
    k = pl.program_id(1)

    @pl.when(k == 0)
    def _init():
        o_ref[...] = jnp.zeros_like(o_ref)

    dims = (((1,), (0,)), ((), ()))
    o_ref[...] += jax.lax.dot_general(
        hi_ref[...], adjc_ref[...], dims, preferred_element_type=jnp.float32
    )

    @pl.when(k == nk - 1)
    def _fin():
        z = u_ref[...] * (o_ref[...] + pself_ref[...]) + b_ref[...]
        if act:
            z = jnp.maximum(z, 0.0)
        o_ref[...] = z


def kernel(x, adj, W1, b1, W2, b2, W3, b3):
    n, feat = x.shape
    hid = W1.shape[1]

    s_blk = 2048
    d_blk = min(1024, ((n + 127) // 128) * 128)
    nk = (n + s_blk - 1) // s_blk
    ni = (n + d_blk - 1) // d_blk
    ns = nk * s_blk

    deg, adjc = pl.pallas_call(
        functools.partial(_deg_cast_kernel, n=n, s_blk=s_blk, nk=nk),
        grid=(ni, nk),
        in_specs=[pl.BlockSpec((s_blk, d_blk), lambda i, k: (k, i))],
        out_specs=[
            pl.BlockSpec((1, d_blk), lambda i, k: (0, i)),
            pl.BlockSpec((s_blk, d_blk), lambda i, k: (k, i)),
        ],
        out_shape=[
            jax.ShapeDtypeStruct((1, n), jnp.float32),
            jax.ShapeDtypeStruct((ns, n), BF),
        ],
    )(adj)

    u = jax.lax.rsqrt(deg)

    proj = pl.pallas_call(
        functools.partial(_proj_kernel, n=n, s_blk=s_blk),
        grid=(nk,),
        in_specs=[
            pl.BlockSpec((feat, s_blk), lambda r: (0, r)),
            pl.BlockSpec((feat, hid), lambda r: (0, 0)),
            pl.BlockSpec((1, s_blk), lambda r: (0, r)),
        ],
        out_specs=[
            pl.BlockSpec((hid, s_blk), lambda r: (0, r)),
            pl.BlockSpec((hid, s_blk), lambda r: (0, r)),
        ],
        out_shape=[
            jax.ShapeDtypeStruct((hid, ns), jnp.float32),
            jax.ShapeDtypeStruct((hid, ns), BF),
        ],
    )

    def conv(pt, hi, b, act):
        return pl.pallas_call(
            functools.partial(_conv_kernel, nk=nk, act=act),
            grid=(ni, nk),
            in_specs=[
                pl.BlockSpec((s_blk, d_blk), lambda i, k: (k, i)),
                pl.BlockSpec((hid, s_blk), lambda i, k: (0, k)),
                pl.BlockSpec((hid, d_blk), lambda i, k: (0, i)),
                pl.BlockSpec((1, d_blk), lambda i, k: (0, i)),
                pl.BlockSpec((hid, 1), lambda i, k: (0, 0)),
            ],
            out_specs=pl.BlockSpec((hid, d_blk), lambda i, k: (0, i)),
            out_shape=jax.ShapeDtypeStruct((hid, n), jnp.float32),
        )(adjc, hi, pt, u, b)

    ht = x.T
    for W, b, act in ((W1, b1, True), (W2, b2, True), (W3, b3, False)):
        pt, hi = proj(ht, W, u)
        ht = conv(pt, hi, b.reshape(hid, 1), act)
    return ht.T


# speedup vs baseline: 1.0922x; 1.0922x over previous
import functools

import jax
import jax.numpy as jnp
from jax.experimental import pallas as pl

I8 = jnp.int8


def _deg_cast_kernel(adj_ref, deg_ref, adjc_ref, *, n, s_blk, nk):
    k = pl.program_id(1)

    @pl.when(k == 0)
    def _init():
        deg_ref[...] = jnp.ones_like(deg_ref)

    @pl.when(k < nk - 1)
    def _full():
        blk = adj_ref[...]
        adjc_ref[...] = blk.astype(I8)
        deg_ref[...] += jnp.sum(blk, axis=0, keepdims=True)

    @pl.when(k == nk - 1)
    def _tail():
        blk = adj_ref[...]
        rows = jax.lax.broadcasted_iota(jnp.int32, blk.shape, 0)
        blk = jnp.where(rows < n - (nk - 1) * s_blk, blk, 0.0)
        adjc_ref[...] = blk.astype(I8)
        deg_ref[...] += jnp.sum(blk, axis=0, keepdims=True)


def _proj_kernel(ht_ref, w_ref, u_ref, pt_ref, hi_ref, lo_ref, scl_ref, *, n, s_blk):
    r = pl.program_id(0)
    pt = u_ref[...] * jax.lax.dot_general(
        w_ref[...],
        ht_ref[...],
        (((0,), (0,)), ((), ())),
        preferred_element_type=jnp.float32,
    )
    cols = jax.lax.broadcasted_iota(jnp.int32, pt.shape, 1)
    pt = jnp.where(cols < n - r * s_blk, pt, 0.0)
    pt_ref[...] = pt
    m = jnp.max(jnp.abs(pt), axis=1, keepdims=True)
    s = jnp.maximum(m, 1e-30) * (1.0 / 32512.0)
    scl_ref[...] = jnp.broadcast_to(s, scl_ref.shape)
    q = jnp.round(pt * (1.0 / s)).astype(jnp.int32)
    hi = jax.lax.shift_right_arithmetic(q + 128, 8)
    hi_ref[...] = hi.astype(I8)
    lo_ref[...] = (q - jax.lax.shift_left(hi, 8)).astype(I8)


def _conv_kernel(adjc_ref, hi_ref, lo_ref, scl_ref, pself_ref, u_ref, b_ref, o_ref, *, nk, act):
    k = pl.program_id(1)

    @pl.when(k == 0)
    def _init():
        o_ref[...] = jnp.zeros_like(o_ref)

    dims = (((1,), (0,)), ((), ()))
    a = adjc_ref[...]
    d = jax.lax.shift_left(
        jax.lax.dot_general(hi_ref[...], a, dims, preferred_element_type=jnp.int32),
        8,
    ) + jax.lax.dot_general(lo_ref[...], a, dims, preferred_element_type=jnp.int32)
    o_ref[...] += scl_ref[:, 0:1] * d.astype(jnp.float32)

    @pl.when(k == nk - 1)
    def _fin():
        z = u_ref[...] * (o_ref[...] + pself_ref[...]) + b_ref[...]
        if act:
            z = jnp.maximum(z, 0.0)
        o_ref[...] = z


def kernel(x, adj, W1, b1, W2, b2, W3, b3):
    n, feat = x.shape
    hid = W1.shape[1]

    s_blk = 2048
    d_blk = min(1024, ((n + 127) // 128) * 128)
    nk = (n + s_blk - 1) // s_blk
    ni = (n + d_blk - 1) // d_blk
    ns = nk * s_blk

    deg, adjc = pl.pallas_call(
        functools.partial(_deg_cast_kernel, n=n, s_blk=s_blk, nk=nk),
        grid=(ni, nk),
        in_specs=[pl.BlockSpec((s_blk, d_blk), lambda i, k: (k, i))],
        out_specs=[
            pl.BlockSpec((1, d_blk), lambda i, k: (0, i)),
            pl.BlockSpec((s_blk, d_blk), lambda i, k: (k, i)),
        ],
        out_shape=[
            jax.ShapeDtypeStruct((1, n), jnp.float32),
            jax.ShapeDtypeStruct((ns, n), I8),
        ],
    )(adj)

    u = jax.lax.rsqrt(deg)

    proj = pl.pallas_call(
        functools.partial(_proj_kernel, n=n, s_blk=s_blk),
        grid=(nk,),
        in_specs=[
            pl.BlockSpec((feat, s_blk), lambda r: (0, r)),
            pl.BlockSpec((feat, hid), lambda r: (0, 0)),
            pl.BlockSpec((1, s_blk), lambda r: (0, r)),
        ],
        out_specs=[
            pl.BlockSpec((hid, s_blk), lambda r: (0, r)),
            pl.BlockSpec((hid, s_blk), lambda r: (0, r)),
            pl.BlockSpec((hid, s_blk), lambda r: (0, r)),
            pl.BlockSpec((hid, 128), lambda r: (0, r)),
        ],
        out_shape=[
            jax.ShapeDtypeStruct((hid, ns), jnp.float32),
            jax.ShapeDtypeStruct((hid, ns), I8),
            jax.ShapeDtypeStruct((hid, ns), I8),
            jax.ShapeDtypeStruct((hid, nk * 128), jnp.float32),
        ],
    )

    def conv(pt, hi, lo, scl, b, act):
        return pl.pallas_call(
            functools.partial(_conv_kernel, nk=nk, act=act),
            grid=(ni, nk),
            in_specs=[
                pl.BlockSpec((s_blk, d_blk), lambda i, k: (k, i)),
                pl.BlockSpec((hid, s_blk), lambda i, k: (0, k)),
                pl.BlockSpec((hid, s_blk), lambda i, k: (0, k)),
                pl.BlockSpec((hid, 128), lambda i, k: (0, k)),
                pl.BlockSpec((hid, d_blk), lambda i, k: (0, i)),
                pl.BlockSpec((1, d_blk), lambda i, k: (0, i)),
                pl.BlockSpec((hid, 1), lambda i, k: (0, 0)),
            ],
            out_specs=pl.BlockSpec((hid, d_blk), lambda i, k: (0, i)),
            out_shape=jax.ShapeDtypeStruct((hid, n), jnp.float32),
        )(adjc, hi, lo, scl, pt, u, b)

    ht = x.T
    for W, b, act in ((W1, b1, True), (W2, b2, True), (W3, b3, False)):
        pt, hi, lo, scl = proj(ht, W, u)
        ht = conv(pt, hi, lo, scl, b.reshape(hid, 1), act)
    return ht.T


# speedup vs baseline: 1.2133x; 1.1109x over previous
import functools

import jax
import jax.numpy as jnp
from jax.experimental import pallas as pl

I8 = jnp.int8
BF16 = jnp.bfloat16


def _deg_cast_kernel(adj_ref, deg_ref, adjc_ref, *, n, s_blk, nk):
    k = pl.program_id(1)

    @pl.when(k == 0)
    def _init():
        deg_ref[...] = jnp.ones_like(deg_ref)

    @pl.when(k < nk - 1)
    def _full():
        blk = adj_ref[...]
        adjc_ref[...] = blk.astype(I8)
        deg_ref[...] += jnp.sum(blk, axis=0, keepdims=True)

    @pl.when(k == nk - 1)
    def _tail():
        blk = adj_ref[...]
        rows = jax.lax.broadcasted_iota(jnp.int32, blk.shape, 0)
        blk = jnp.where(rows < n - (nk - 1) * s_blk, blk, 0.0)
        adjc_ref[...] = blk.astype(I8)
        deg_ref[...] += jnp.sum(blk, axis=0, keepdims=True)


def _proj_kernel(ht_ref, w_ref, u_ref, pt_ref, ptb_ref, *, n, s_blk):
    r = pl.program_id(0)
    pt = u_ref[...] * jax.lax.dot_general(
        w_ref[...],
        ht_ref[...],
        (((0,), (0,)), ((), ())),
        preferred_element_type=jnp.float32,
    )
    cols = jax.lax.broadcasted_iota(jnp.int32, pt.shape, 1)
    pt = jnp.where(cols < n - r * s_blk, pt, 0.0)
    pt_ref[...] = pt
    ptb_ref[...] = pt.astype(BF16)


def _conv_kernel(adjc_ref, ptb_ref, pself_ref, u_ref, b_ref, o_ref, *, nk, act):
    k = pl.program_id(1)

    @pl.when(k == 0)
    def _init():
        o_ref[...] = jnp.zeros_like(o_ref)

    o_ref[...] += jax.lax.dot_general(
        ptb_ref[...],
        adjc_ref[...].astype(BF16),
        (((1,), (0,)), ((), ())),
        preferred_element_type=jnp.float32,
    )

    @pl.when(k == nk - 1)
    def _fin():
        z = u_ref[...] * (o_ref[...] + pself_ref[...]) + b_ref[...]
        if act:
            z = jnp.maximum(z, 0.0)
        o_ref[...] = z


def kernel(x, adj, W1, b1, W2, b2, W3, b3):
    n, feat = x.shape
    hid = W1.shape[1]

    s_blk = 2048
    d_blk = min(1024, ((n + 127) // 128) * 128)
    nk = (n + s_blk - 1) // s_blk
    ni = (n + d_blk - 1) // d_blk
    ns = nk * s_blk

    deg, adjc = pl.pallas_call(
        functools.partial(_deg_cast_kernel, n=n, s_blk=s_blk, nk=nk),
        grid=(ni, nk),
        in_specs=[pl.BlockSpec((s_blk, d_blk), lambda i, k: (k, i))],
        out_specs=[
            pl.BlockSpec((1, d_blk), lambda i, k: (0, i)),
            pl.BlockSpec((s_blk, d_blk), lambda i, k: (k, i)),
        ],
        out_shape=[
            jax.ShapeDtypeStruct((1, n), jnp.float32),
            jax.ShapeDtypeStruct((ns, n), I8),
        ],
    )(adj)

    u = jax.lax.rsqrt(deg)

    proj = pl.pallas_call(
        functools.partial(_proj_kernel, n=n, s_blk=s_blk),
        grid=(nk,),
        in_specs=[
            pl.BlockSpec((feat, s_blk), lambda r: (0, r)),
            pl.BlockSpec((feat, hid), lambda r: (0, 0)),
            pl.BlockSpec((1, s_blk), lambda r: (0, r)),
        ],
        out_specs=[
            pl.BlockSpec((hid, s_blk), lambda r: (0, r)),
            pl.BlockSpec((hid, s_blk), lambda r: (0, r)),
        ],
        out_shape=[
            jax.ShapeDtypeStruct((hid, ns), jnp.float32),
            jax.ShapeDtypeStruct((hid, ns), BF16),
        ],
    )

    def conv(pt, ptb, b, act):
        return pl.pallas_call(
            functools.partial(_conv_kernel, nk=nk, act=act),
            grid=(ni, nk),
            in_specs=[
                pl.BlockSpec((s_blk, d_blk), lambda i, k: (k, i)),
                pl.BlockSpec((hid, s_blk), lambda i, k: (0, k)),
                pl.BlockSpec((hid, d_blk), lambda i, k: (0, i)),
                pl.BlockSpec((1, d_blk), lambda i, k: (0, i)),
                pl.BlockSpec((hid, 1), lambda i, k: (0, 0)),
            ],
            out_specs=pl.BlockSpec((hid, d_blk), lambda i, k: (0, i)),
            out_shape=jax.ShapeDtypeStruct((hid, n), jnp.float32),
        )(adjc, ptb, pt, u, b)

    ht = x.T
    for W, b, act in ((W1, b1, True), (W2, b2, True), (W3, b3, False)):
        pt, ptb = proj(ht, W, u)
        ht = conv(pt, ptb, b.reshape(hid, 1), act)
    return ht.T


# speedup vs baseline: 1.4045x; 1.1575x over previous
import functools

import jax
import jax.numpy as jnp
from jax.experimental import pallas as pl

I8 = jnp.int8
BF16 = jnp.bfloat16


def _deg_cast_kernel(adj_ref, deg_ref, adjc_ref, *, n, s_blk, nk):
    k = pl.program_id(1)

    @pl.when(k == 0)
    def _init():
        deg_ref[...] = jnp.ones_like(deg_ref)

    @pl.when(k < nk - 1)
    def _full():
        blk = adj_ref[...]
        adjc_ref[...] = blk.astype(I8)
        deg_ref[...] += jnp.sum(blk, axis=0, keepdims=True)

    @pl.when(k == nk - 1)
    def _tail():
        blk = adj_ref[...]
        rows = jax.lax.broadcasted_iota(jnp.int32, blk.shape, 0)
        blk = jnp.where(rows < n - (nk - 1) * s_blk, blk, 0.0)
        adjc_ref[...] = blk.astype(I8)
        deg_ref[...] += jnp.sum(blk, axis=0, keepdims=True)


def _proj_kernel(ht_ref, w_ref, u_ref, pt_ref, ptb_ref, *, n, s_blk):
    r = pl.program_id(0)
    pt = u_ref[...] * jax.lax.dot_general(
        w_ref[...],
        ht_ref[...],
        (((0,), (0,)), ((), ())),
        preferred_element_type=jnp.float32,
    )
    cols = jax.lax.broadcasted_iota(jnp.int32, pt.shape, 1)
    pt = jnp.where(cols < n - r * s_blk, pt, 0.0)
    pt_ref[...] = pt
    ptb_ref[...] = pt.astype(BF16)


def _conv_kernel(adjc_ref, ptb_ref, pself_ref, u_ref, b_ref, o_ref, *, nk, act):
    k = pl.program_id(1)

    @pl.when(k == 0)
    def _init():
        o_ref[...] = jnp.zeros_like(o_ref)

    o_ref[...] += jax.lax.dot_general(
        ptb_ref[...],
        adjc_ref[...].astype(BF16),
        (((1,), (0,)), ((), ())),
        preferred_element_type=jnp.float32,
    )

    @pl.when(k == nk - 1)
    def _fin():
        z = u_ref[...] * (o_ref[...] + pself_ref[...]) + b_ref[...]
        if act:
            z = jnp.maximum(z, 0.0)
        o_ref[...] = z


def kernel(x, adj, W1, b1, W2, b2, W3, b3):
    n, feat = x.shape
    hid = W1.shape[1]

    s_blk = 2048
    d_blk = min(2048, ((n + 127) // 128) * 128)
    nk = (n + s_blk - 1) // s_blk
    ni = (n + d_blk - 1) // d_blk
    ns = nk * s_blk

    deg, adjc = pl.pallas_call(
        functools.partial(_deg_cast_kernel, n=n, s_blk=s_blk, nk=nk),
        grid=(ni, nk),
        in_specs=[pl.BlockSpec((s_blk, d_blk), lambda i, k: (k, i))],
        out_specs=[
            pl.BlockSpec((1, d_blk), lambda i, k: (0, i)),
            pl.BlockSpec((s_blk, d_blk), lambda i, k: (k, i)),
        ],
        out_shape=[
            jax.ShapeDtypeStruct((1, n), jnp.float32),
            jax.ShapeDtypeStruct((ns, n), I8),
        ],
    )(adj)

    u = jax.lax.rsqrt(deg)

    proj = pl.pallas_call(
        functools.partial(_proj_kernel, n=n, s_blk=s_blk),
        grid=(nk,),
        in_specs=[
            pl.BlockSpec((feat, s_blk), lambda r: (0, r)),
            pl.BlockSpec((feat, hid), lambda r: (0, 0)),
            pl.BlockSpec((1, s_blk), lambda r: (0, r)),
        ],
        out_specs=[
            pl.BlockSpec((hid, s_blk), lambda r: (0, r)),
            pl.BlockSpec((hid, s_blk), lambda r: (0, r)),
        ],
        out_shape=[
            jax.ShapeDtypeStruct((hid, ns), jnp.float32),
            jax.ShapeDtypeStruct((hid, ns), BF16),
        ],
    )

    def conv(pt, ptb, b, act):
        return pl.pallas_call(
            functools.partial(_conv_kernel, nk=nk, act=act),
            grid=(ni, nk),
            in_specs=[
                pl.BlockSpec((s_blk, d_blk), lambda i, k: (k, i)),
                pl.BlockSpec((hid, s_blk), lambda i, k: (0, k)),
                pl.BlockSpec((hid, d_blk), lambda i, k: (0, i)),
                pl.BlockSpec((1, d_blk), lambda i, k: (0, i)),
                pl.BlockSpec((hid, 1), lambda i, k: (0, 0)),
            ],
            out_specs=pl.BlockSpec((hid, d_blk), lambda i, k: (0, i)),
            out_shape=jax.ShapeDtypeStruct((hid, n), jnp.float32),
        )(adjc, ptb, pt, u, b)

    ht = x.T
    for W, b, act in ((W1, b1, True), (W2, b2, True), (W3, b3, False)):
        pt, ptb = proj(ht, W, u)
        ht = conv(pt, ptb, b.reshape(hid, 1), act)
    return ht.T


# speedup vs baseline: 1.4420x; 1.0267x over previous
import functools

import jax
import jax.numpy as jnp
from jax.experimental import pallas as pl

I8 = jnp.int8
BF16 = jnp.bfloat16


def _deg_cast_kernel(adj_ref, deg_ref, adjc_ref, *, n, s_blk, nk):
    k = pl.program_id(1)

    @pl.when(k == 0)
    def _init():
        deg_ref[...] = jnp.ones_like(deg_ref)

    @pl.when(k < nk - 1)
    def _full():
        blk = adj_ref[...]
        adjc_ref[...] = blk.astype(I8)
        deg_ref[...] += jnp.sum(blk, axis=0, keepdims=True)

    @pl.when(k == nk - 1)
    def _tail():
        blk = adj_ref[...]
        rows = jax.lax.broadcasted_iota(jnp.int32, blk.shape, 0)
        blk = jnp.where(rows < n - (nk - 1) * s_blk, blk, 0.0)
        adjc_ref[...] = blk.astype(I8)
        deg_ref[...] += jnp.sum(blk, axis=0, keepdims=True)


def _proj_kernel(ht_ref, w_ref, u_ref, pt_ref, ptb_ref, *, n, s_blk):
    r = pl.program_id(0)
    pt = u_ref[...] * jax.lax.dot_general(
        w_ref[...],
        ht_ref[...],
        (((0,), (0,)), ((), ())),
        preferred_element_type=jnp.float32,
    )
    cols = jax.lax.broadcasted_iota(jnp.int32, pt.shape, 1)
    pt = jnp.where(cols < n - r * s_blk, pt, 0.0)
    pt_ref[...] = pt
    ptb_ref[...] = pt.astype(BF16)


def _conv_kernel(adjc_ref, ptb_ref, pself_ref, u_ref, b_ref, o_ref, *, nk, act):
    k = pl.program_id(1)

    @pl.when(k == 0)
    def _init():
        o_ref[...] = jnp.zeros_like(o_ref)

    o_ref[...] += jax.lax.dot_general(
        ptb_ref[...],
        adjc_ref[...].astype(BF16),
        (((1,), (0,)), ((), ())),
        preferred_element_type=jnp.float32,
    )

    @pl.when(k == nk - 1)
    def _fin():
        z = u_ref[...] * (o_ref[...] + pself_ref[...]) + b_ref[...]
        if act:
            z = jnp.maximum(z, 0.0)
        o_ref[...] = z


def kernel(x, adj, W1, b1, W2, b2, W3, b3):
    n, feat = x.shape
    hid = W1.shape[1]

    s_blk = 2048
    d_blk = min(2560, ((n + 127) // 128) * 128)
    nk = (n + s_blk - 1) // s_blk
    ni = (n + d_blk - 1) // d_blk
    ns = nk * s_blk

    deg, adjc = pl.pallas_call(
        functools.partial(_deg_cast_kernel, n=n, s_blk=s_blk, nk=nk),
        grid=(ni, nk),
        in_specs=[pl.BlockSpec((s_blk, d_blk), lambda i, k: (k, i))],
        out_specs=[
            pl.BlockSpec((1, d_blk), lambda i, k: (0, i)),
            pl.BlockSpec((s_blk, d_blk), lambda i, k: (k, i)),
        ],
        out_shape=[
            jax.ShapeDtypeStruct((1, n), jnp.float32),
            jax.ShapeDtypeStruct((ns, n), I8),
        ],
    )(adj)

    u = jax.lax.rsqrt(deg)

    proj = pl.pallas_call(
        functools.partial(_proj_kernel, n=n, s_blk=s_blk),
        grid=(nk,),
        in_specs=[
            pl.BlockSpec((feat, s_blk), lambda r: (0, r)),
            pl.BlockSpec((feat, hid), lambda r: (0, 0)),
            pl.BlockSpec((1, s_blk), lambda r: (0, r)),
        ],
        out_specs=[
            pl.BlockSpec((hid, s_blk), lambda r: (0, r)),
            pl.BlockSpec((hid, s_blk), lambda r: (0, r)),
        ],
        out_shape=[
            jax.ShapeDtypeStruct((hid, ns), jnp.float32),
            jax.ShapeDtypeStruct((hid, ns), BF16),
        ],
    )

    def conv(pt, ptb, b, act):
        return pl.pallas_call(
            functools.partial(_conv_kernel, nk=nk, act=act),
            grid=(ni, nk),
            in_specs=[
                pl.BlockSpec((s_blk, d_blk), lambda i, k: (k, i)),
                pl.BlockSpec((hid, s_blk), lambda i, k: (0, k)),
                pl.BlockSpec((hid, d_blk), lambda i, k: (0, i)),
                pl.BlockSpec((1, d_blk), lambda i, k: (0, i)),
                pl.BlockSpec((hid, 1), lambda i, k: (0, 0)),
            ],
            out_specs=pl.BlockSpec((hid, d_blk), lambda i, k: (0, i)),
            out_shape=jax.ShapeDtypeStruct((hid, n), jnp.float32),
        )(adjc, ptb, pt, u, b)

    ht = x.T
    for W, b, act in ((W1, b1, True), (W2, b2, True), (W3, b3, False)):
        pt, ptb = proj(ht, W, u)
        ht = conv(pt, ptb, b.reshape(hid, 1), act)
    return ht.T


# speedup vs baseline: 1.6314x; 1.1313x over previous
import functools

import jax
import jax.numpy as jnp
from jax.experimental import pallas as pl

I8 = jnp.int4
BF16 = jnp.bfloat16


def _deg_cast_kernel(adj_ref, deg_ref, adjc_ref, *, n, s_blk, nk):
    k = pl.program_id(1)

    @pl.when(k == 0)
    def _init():
        deg_ref[...] = jnp.ones_like(deg_ref)

    @pl.when(k < nk - 1)
    def _full():
        blk = adj_ref[...]
        adjc_ref[...] = blk.astype(I8)
        deg_ref[...] += jnp.sum(blk, axis=0, keepdims=True)

    @pl.when(k == nk - 1)
    def _tail():
        blk = adj_ref[...]
        rows = jax.lax.broadcasted_iota(jnp.int32, blk.shape, 0)
        blk = jnp.where(rows < n - (nk - 1) * s_blk, blk, 0.0)
        adjc_ref[...] = blk.astype(I8)
        deg_ref[...] += jnp.sum(blk, axis=0, keepdims=True)


def _proj_kernel(ht_ref, w_ref, u_ref, pt_ref, ptb_ref, *, n, s_blk):
    r = pl.program_id(0)
    pt = u_ref[...] * jax.lax.dot_general(
        w_ref[...],
        ht_ref[...],
        (((0,), (0,)), ((), ())),
        preferred_element_type=jnp.float32,
    )
    cols = jax.lax.broadcasted_iota(jnp.int32, pt.shape, 1)
    pt = jnp.where(cols < n - r * s_blk, pt, 0.0)
    pt_ref[...] = pt
    ptb_ref[...] = pt.astype(BF16)


def _conv_kernel(adjc_ref, ptb_ref, pself_ref, u_ref, b_ref, o_ref, *, nk, act):
    k = pl.program_id(1)

    @pl.when(k == 0)
    def _init():
        o_ref[...] = jnp.zeros_like(o_ref)

    o_ref[...] += jax.lax.dot_general(
        ptb_ref[...],
        adjc_ref[...].astype(BF16),
        (((1,), (0,)), ((), ())),
        preferred_element_type=jnp.float32,
    )

    @pl.when(k == nk - 1)
    def _fin():
        z = u_ref[...] * (o_ref[...] + pself_ref[...]) + b_ref[...]
        if act:
            z = jnp.maximum(z, 0.0)
        o_ref[...] = z


def kernel(x, adj, W1, b1, W2, b2, W3, b3):
    n, feat = x.shape
    hid = W1.shape[1]

    s_blk = 2048
    d_blk = min(2560, ((n + 127) // 128) * 128)
    nk = (n + s_blk - 1) // s_blk
    ni = (n + d_blk - 1) // d_blk
    ns = nk * s_blk

    deg, adjc = pl.pallas_call(
        functools.partial(_deg_cast_kernel, n=n, s_blk=s_blk, nk=nk),
        grid=(ni, nk),
        in_specs=[pl.BlockSpec((s_blk, d_blk), lambda i, k: (k, i))],
        out_specs=[
            pl.BlockSpec((1, d_blk), lambda i, k: (0, i)),
            pl.BlockSpec((s_blk, d_blk), lambda i, k: (k, i)),
        ],
        out_shape=[
            jax.ShapeDtypeStruct((1, n), jnp.float32),
            jax.ShapeDtypeStruct((ns, n), I8),
        ],
    )(adj)

    u = jax.lax.rsqrt(deg)

    proj = pl.pallas_call(
        functools.partial(_proj_kernel, n=n, s_blk=s_blk),
        grid=(nk,),
        in_specs=[
            pl.BlockSpec((feat, s_blk), lambda r: (0, r)),
            pl.BlockSpec((feat, hid), lambda r: (0, 0)),
            pl.BlockSpec((1, s_blk), lambda r: (0, r)),
        ],
        out_specs=[
            pl.BlockSpec((hid, s_blk), lambda r: (0, r)),
            pl.BlockSpec((hid, s_blk), lambda r: (0, r)),
        ],
        out_shape=[
            jax.ShapeDtypeStruct((hid, ns), jnp.float32),
            jax.ShapeDtypeStruct((hid, ns), BF16),
        ],
    )

    def conv(pt, ptb, b, act):
        return pl.pallas_call(
            functools.partial(_conv_kernel, nk=nk, act=act),
            grid=(ni, nk),
            in_specs=[
                pl.BlockSpec((s_blk, d_blk), lambda i, k: (k, i)),
                pl.BlockSpec((hid, s_blk), lambda i, k: (0, k)),
                pl.BlockSpec((hid, d_blk), lambda i, k: (0, i)),
                pl.BlockSpec((1, d_blk), lambda i, k: (0, i)),
                pl.BlockSpec((hid, 1), lambda i, k: (0, 0)),
            ],
            out_specs=pl.BlockSpec((hid, d_blk), lambda i, k: (0, i)),
            out_shape=jax.ShapeDtypeStruct((hid, n), jnp.float32),
        )(adjc, ptb, pt, u, b)

    ht = x.T
    for W, b, act in ((W1, b1, True), (W2, b2, True), (W3, b3, False)):
        pt, ptb = proj(ht, W, u)
        ht = conv(pt, ptb, b.reshape(hid, 1), act)
    return ht.T
